# baseline (speedup 1.0000x reference)
import functools

import jax
import jax.numpy as jnp
from jax import lax
from jax.experimental import pallas as pl
from jax.experimental.pallas import tpu as pltpu
from jax.experimental.pallas import tpu_sc as plsc

N = 10000
E = 320000
DF = 128
DH = 128
DO = 64
DE = 16
G = 64

HW = DH // 2
NTILES = 32
GRP = 256
C = 40
EPAD = NTILES * C * GRP
NPAD = 10240
RPT = NPAD // 16
BLK = 1000
NB = N // BLK

_HIGH = jax.lax.Precision.HIGHEST


def _dot(a, b):
    return jax.lax.dot(a, b, precision=_HIGH, preferred_element_type=jnp.float32)


def _prelu(v, a):
    return jnp.where(v >= 0, v, a * v)


_NBUF = 4


def _sc_scatter(hn, srcb, dstb, z, outp, src_v, dst_v, rows, acc, gs, ss):
    c = lax.axis_index("c")
    s = lax.axis_index("s")
    b = c * 16 + s
    pltpu.sync_copy(srcb.at[b], src_v)
    pltpu.sync_copy(dstb.at[b], dst_v)
    r0 = s * RPT
    pltpu.sync_copy(z, acc.at[pl.ds(r0, RPT)])

    def start_rows(g, p):
        pltpu.async_copy(hn.at[src_v.at[g]], rows[p], gs[p])

    def wait_rows(p):
        pltpu.make_async_copy(hn.at[src_v.at[0]], rows[p], gs[p]).wait()

    def start_scat(g, p):
        pltpu.async_copy(rows[p], acc.at[dst_v.at[g]], ss[p], add=True)

    def wait_scat(p):
        pltpu.make_async_copy(rows[p], acc.at[dst_v.at[0]], ss[p]).wait()

    start_rows(0, 0)
    start_rows(1, 1)
    plsc.subcore_barrier()

    def phase(g, p):
        wait_rows(p)

        @pl.when(g >= 2)
        def _():
            wait_scat((p - 2) % _NBUF)

        start_scat(g, p)

        @pl.when(g + 2 < C)
        def _():
            start_rows(g + 2, (p + 2) % _NBUF)

    def body(i, carry):
        j = i * _NBUF
        phase(j, 0)

        @pl.when(j + 1 < C)
        def _():
            phase(j + 1, 1)

        @pl.when(j + 2 < C)
        def _():
            phase(j + 2, 2)

        @pl.when(j + 3 < C)
        def _():
            phase(j + 3, 3)

        return carry

    lax.fori_loop(0, (C + _NBUF - 1) // _NBUF, body, 0)
    wait_scat((C - 1) % _NBUF)
    wait_scat((C - 2) % _NBUF)
    plsc.subcore_barrier()
    pltpu.sync_copy(acc.at[pl.ds(r0, RPT)], outp.at[c, pl.ds(r0, RPT)])


def _sc_ea(ea, dstb, z16, out16, dst_v, eab, acc16, egs, ess):
    c = lax.axis_index("c")
    s = lax.axis_index("s")
    b = c * 16 + s
    pltpu.sync_copy(dstb.at[b], dst_v)
    r0 = s * RPT
    pltpu.sync_copy(z16, acc16.at[pl.ds(r0, RPT)])
    ebase = b * (C * GRP)

    def start_ea(g, p):
        pltpu.async_copy(ea.at[pl.ds(ebase + g * GRP, GRP)], eab[p], egs[p])

    def wait_ea(p):
        pltpu.make_async_copy(ea.at[pl.ds(0, GRP)], eab[p], egs[p]).wait()

    def start_escat(g, p):
        pltpu.async_copy(eab[p], acc16.at[dst_v.at[g]], ess[p], add=True)

    def wait_escat(p):
        pltpu.make_async_copy(eab[p], acc16.at[dst_v.at[0]], ess[p]).wait()

    start_ea(0, 0)
    start_ea(1, 1)
    plsc.subcore_barrier()

    def phase(g, p):
        wait_ea(p)

        @pl.when(g >= 2)
        def _():
            wait_escat((p - 2) % _NBUF)

        start_escat(g, p)

        @pl.when(g + 2 < C)
        def _():
            start_ea(g + 2, (p + 2) % _NBUF)

    def body(i, carry):
        j = i * _NBUF
        phase(j, 0)

        @pl.when(j + 1 < C)
        def _():
            phase(j + 1, 1)

        @pl.when(j + 2 < C)
        def _():
            phase(j + 2, 2)

        @pl.when(j + 3 < C)
        def _():
            phase(j + 3, 3)

        return carry

    lax.fori_loop(0, (C + _NBUF - 1) // _NBUF, body, 0)
    wait_escat((C - 1) % _NBUF)
    wait_escat((C - 2) % _NBUF)
    plsc.subcore_barrier()
    pltpu.sync_copy(acc16.at[pl.ds(r0, RPT)], out16.at[c, pl.ds(r0, RPT)])


@functools.lru_cache(maxsize=None)
def _sc_calls():
    mesh = plsc.VectorSubcoreMesh(core_axis_name="c", subcore_axis_name="s")
    cparams = pltpu.CompilerParams(use_tc_tiling_on_sc=False)
    ea_call = functools.partial(
        pl.kernel,
        out_type=jax.ShapeDtypeStruct((2, NPAD, DE), jnp.float32),
        mesh=mesh,
        compiler_params=cparams,
        scratch_types=[
            pltpu.VMEM((C, GRP), jnp.int32),
            [pltpu.VMEM((GRP, DE), jnp.float32)] * _NBUF,
            pltpu.VMEM_SHARED((NPAD, DE), jnp.float32),
            [pltpu.SemaphoreType.DMA] * _NBUF,
            [pltpu.SemaphoreType.DMA] * _NBUF,
        ],
    )(_sc_ea)
    scatter = functools.partial(
        pl.kernel,
        out_type=jax.ShapeDtypeStruct((2, NPAD, DH), jnp.bfloat16),
        mesh=mesh,
        compiler_params=cparams,
        scratch_types=[
            pltpu.VMEM((C, GRP), jnp.int32),
            pltpu.VMEM((C, GRP), jnp.int32),
            [pltpu.VMEM((GRP, DH), jnp.bfloat16)] * _NBUF,
            pltpu.VMEM_SHARED((NPAD, DH), jnp.bfloat16),
            [pltpu.SemaphoreType.DMA] * _NBUF,
            [pltpu.SemaphoreType.DMA] * _NBUF,
        ],
    )(_sc_scatter)
    return ea_call, scatter


def _agg(p_ref):
    return (p_ref[0].astype(jnp.float32) + p_ref[1].astype(jnp.float32))


def _tc1_body(x_ref, w1_ref, b1_ref, a1_ref, wn_ref, h0_ref, hn_ref):
    h = _dot(x_ref[...], w1_ref[...]) + b1_ref[...]
    h = _prelu(h, a1_ref[0, 0])
    h0_ref[...] = h
    hn_ref[...] = _dot(h, wn_ref[...]).astype(jnp.bfloat16)


def _tc2_body(h_ref, p_ref, e_ref, wr_ref, we_ref, b_ref, a_ref, wn_ref,
              h1_ref, hn_ref):
    eagg = e_ref[0] + e_ref[1]
    v = (_dot(h_ref[...], wr_ref[...]) + _agg(p_ref)
         + _dot(eagg, we_ref[...]) + b_ref[...])
    h1 = _prelu(v, a_ref[0, 0])
    h1_ref[...] = h1
    hn_ref[...] = _dot(h1, wn_ref[...]).astype(jnp.bfloat16)


def _tc3_body(h_ref, p_ref, e_ref, wr_ref, we_ref, b_ref, a_ref, bat_ref,
              w2_ref, b2_ref, out_ref, acc_ref, cnt_ref):
    i = pl.program_id(0)
    eagg = e_ref[0] + e_ref[1]
    v = (_dot(h_ref[...], wr_ref[...]) + _agg(p_ref)
         + _dot(eagg, we_ref[...]) + b_ref[...])
    h2 = _prelu(v, a_ref[0, 0])

    bb = bat_ref[0, 0, :]
    gi = lax.broadcasted_iota(jnp.int32, (G, BLK), 0)
    oh = (bb[None, :] == gi).astype(jnp.float32)

    @pl.when(i == 0)
    def _():
        acc_ref[...] = jnp.zeros((G, DH), jnp.float32)
        cnt_ref[...] = jnp.zeros((G, DH), jnp.float32)

    acc_ref[...] += _dot(oh, h2)
    cnt_ref[...] += jnp.broadcast_to(jnp.sum(oh, axis=1, keepdims=True), (G, DH))

    @pl.when(i == NB - 1)
    def _():
        sums = acc_ref[...]
        counts = cnt_ref[...][:, :1]
        hg = sums / jnp.maximum(counts, 1.0)
        nrm = jnp.sqrt(jnp.sum(hg * hg, axis=1, keepdims=True))
        hg = hg / jnp.maximum(nrm, 1e-12)
        out_ref[...] = _dot(hg, w2_ref[...]) + b2_ref[...]


def _full(shape):
    return pl.BlockSpec(shape, lambda i: tuple(0 for _ in shape))


def _rows(width):
    return pl.BlockSpec((BLK, width), lambda i: (i, 0))


_PSPEC = pl.BlockSpec((2, BLK, DH), lambda i: (0, i, 0))
_ESPEC = pl.BlockSpec((2, BLK, DE), lambda i: (0, i, 0))


def _tc1(x, w1, b1, a1, wn):
    return pl.pallas_call(
        _tc1_body,
        grid=(NB,),
        in_specs=[_rows(DF), _full((DF, DH)), _full((1, DH)), _full((1, 1)),
                  _full((DH, DH))],
        out_specs=[_rows(DH), _rows(DH)],
        out_shape=[jax.ShapeDtypeStruct((N, DH), jnp.float32),
                   jax.ShapeDtypeStruct((N, DH), jnp.bfloat16)],
    )(x, w1, b1, a1, wn)


def _tc2(h, p, e, wr, we, b, a, wn):
    return pl.pallas_call(
        _tc2_body,
        grid=(NB,),
        in_specs=[_rows(DH), _PSPEC, _ESPEC, _full((DH, DH)),
                  _full((DE, DH)), _full((1, DH)), _full((1, 1)),
                  _full((DH, DH))],
        out_specs=[_rows(DH), _rows(DH)],
        out_shape=[jax.ShapeDtypeStruct((N, DH), jnp.float32),
                   jax.ShapeDtypeStruct((N, DH), jnp.bfloat16)],
    )(h, p, e, wr, we, b, a, wn)


def _tc3(h, p, e, wr, we, b, a, bat, w2, b2):
    return pl.pallas_call(
        _tc3_body,
        grid=(NB,),
        in_specs=[_rows(DH), _PSPEC, _ESPEC, _full((DH, DH)),
                  _full((DE, DH)), _full((1, DH)), _full((1, 1)),
                  pl.BlockSpec((1, 1, BLK), lambda i: (i, 0, 0)),
                  _full((DH, DO)), _full((1, DO))],
        out_specs=_full((G, DO)),
        out_shape=jax.ShapeDtypeStruct((G, DO), jnp.float32),
        scratch_shapes=[pltpu.VMEM((G, DH), jnp.float32),
                        pltpu.VMEM((G, DH), jnp.float32)],
    )(h, p, e, wr, we, b, a, bat, w2, b2)


def kernel(x, edge_index, edge_attr, batch, fc1_W, fc1_b, a_fc1,
           gc1_Wr, gc1_Wn, gc1_We, gc1_b, a_gc1,
           gc2_Wr, gc2_Wn, gc2_We, gc2_b, a_gc2,
           fc2_W, fc2_b):
    src = edge_index[0]
    dst = edge_index[1]
    pad = EPAD - E
    src_b = jnp.concatenate(
        [src, jnp.zeros((pad,), jnp.int32)]).reshape(NTILES, C, GRP)
    dst_b = jnp.concatenate(
        [dst, jnp.full((pad,), NPAD - 1, jnp.int32)]).reshape(NTILES, C, GRP)
    ea_p = jnp.concatenate([edge_attr, jnp.zeros((pad, DE), jnp.float32)])
    zb = jnp.zeros((RPT, DH), jnp.bfloat16)
    z16 = jnp.zeros((RPT, DE), jnp.float32)
    bat3 = batch.reshape(NB, 1, BLK)

    b1 = fc1_b.reshape(1, DH)
    bg1 = gc1_b.reshape(1, DH)
    bg2 = gc2_b.reshape(1, DH)
    b2 = fc2_b.reshape(1, DO)
    a1 = a_fc1.reshape(1, 1)
    ag1 = a_gc1.reshape(1, 1)
    ag2 = a_gc2.reshape(1, 1)

    ea_call, scatter_call = _sc_calls()
    e16 = ea_call(ea_p, dst_b, z16)
    h0, hn1 = _tc1(x, fc1_W, b1, a1, gc1_Wn)
    p1 = scatter_call(hn1, src_b, dst_b, zb)
    h1, hn2 = _tc2(h0, p1, e16, gc1_Wr, gc1_We, bg1, ag1, gc2_Wn)
    p2 = scatter_call(hn2, src_b, dst_b, zb)
    out = _tc3(h1, p2, e16, gc2_Wr, gc2_We, bg2, ag2, bat3, fc2_W, b2)
    return out


# speedup vs baseline: 1.2207x; 1.2207x over previous
import functools

import jax
import jax.numpy as jnp
from jax import lax
from jax.experimental import pallas as pl
from jax.experimental.pallas import tpu as pltpu
from jax.experimental.pallas import tpu_sc as plsc

N = 10000
E = 320000
DF = 128
DH = 128
DO = 64
DE = 16
G = 64

HW = DH // 2
NTILES = 32
GRP = 128
C = 79
EPAD = NTILES * C * GRP
NPAD = 10240
RPT = NPAD // 16
BLK = 1000
NB = N // BLK

_HIGH = jax.lax.Precision.HIGHEST


def _dot(a, b):
    return jax.lax.dot(a, b, precision=_HIGH, preferred_element_type=jnp.float32)


def _prelu(v, a):
    return jnp.where(v >= 0, v, a * v)


_NBUF = 4


def _sc_scatter(hn, srcb, dstb, z, outp, src_v, dst_v, rows, acc, gs, ss):
    c = lax.axis_index("c")
    s = lax.axis_index("s")
    b = c * 16 + s
    pltpu.sync_copy(srcb.at[b], src_v)
    pltpu.sync_copy(dstb.at[b], dst_v)
    r0 = s * RPT
    pltpu.sync_copy(z, acc.at[pl.ds(r0, RPT)])

    def start_rows(g, p):
        pltpu.async_copy(hn.at[src_v.at[g]], rows[p], gs[p])

    def wait_rows(p):
        pltpu.make_async_copy(hn.at[src_v.at[0]], rows[p], gs[p]).wait()

    def start_scat(g, p):
        pltpu.async_copy(rows[p], acc.at[dst_v.at[g]], ss[p], add=True)

    def wait_scat(p):
        pltpu.make_async_copy(rows[p], acc.at[dst_v.at[0]], ss[p]).wait()

    start_rows(0, 0)
    start_rows(1, 1)
    plsc.subcore_barrier()

    def phase(g, p):
        wait_rows(p)

        @pl.when(g >= 2)
        def _():
            wait_scat((p - 2) % _NBUF)

        start_scat(g, p)

        @pl.when(g + 2 < C)
        def _():
            start_rows(g + 2, (p + 2) % _NBUF)

    def body(i, carry):
        j = i * _NBUF
        phase(j, 0)

        @pl.when(j + 1 < C)
        def _():
            phase(j + 1, 1)

        @pl.when(j + 2 < C)
        def _():
            phase(j + 2, 2)

        @pl.when(j + 3 < C)
        def _():
            phase(j + 3, 3)

        return carry

    lax.fori_loop(0, (C + _NBUF - 1) // _NBUF, body, 0)
    wait_scat((C - 1) % _NBUF)
    wait_scat((C - 2) % _NBUF)
    plsc.subcore_barrier()
    pltpu.sync_copy(acc.at[pl.ds(r0, RPT)], outp.at[c, pl.ds(r0, RPT)])


def _sc_ea(ea, dstb, z16, out16, dst_v, eab, acc16, egs, ess):
    c = lax.axis_index("c")
    s = lax.axis_index("s")
    b = c * 16 + s
    pltpu.sync_copy(dstb.at[b], dst_v)
    r0 = s * RPT
    pltpu.sync_copy(z16, acc16.at[pl.ds(r0, RPT)])
    ebase = b * (C * GRP)

    def start_ea(g, p):
        pltpu.async_copy(ea.at[pl.ds(ebase + g * GRP, GRP)], eab[p], egs[p])

    def wait_ea(p):
        pltpu.make_async_copy(ea.at[pl.ds(0, GRP)], eab[p], egs[p]).wait()

    def start_escat(g, p):
        pltpu.async_copy(eab[p], acc16.at[dst_v.at[g]], ess[p], add=True)

    def wait_escat(p):
        pltpu.make_async_copy(eab[p], acc16.at[dst_v.at[0]], ess[p]).wait()

    start_ea(0, 0)
    start_ea(1, 1)
    plsc.subcore_barrier()

    def phase(g, p):
        wait_ea(p)

        @pl.when(g >= 2)
        def _():
            wait_escat((p - 2) % _NBUF)

        start_escat(g, p)

        @pl.when(g + 2 < C)
        def _():
            start_ea(g + 2, (p + 2) % _NBUF)

    def body(i, carry):
        j = i * _NBUF
        phase(j, 0)

        @pl.when(j + 1 < C)
        def _():
            phase(j + 1, 1)

        @pl.when(j + 2 < C)
        def _():
            phase(j + 2, 2)

        @pl.when(j + 3 < C)
        def _():
            phase(j + 3, 3)

        return carry

    lax.fori_loop(0, (C + _NBUF - 1) // _NBUF, body, 0)
    wait_escat((C - 1) % _NBUF)
    wait_escat((C - 2) % _NBUF)
    plsc.subcore_barrier()
    pltpu.sync_copy(acc16.at[pl.ds(r0, RPT)], out16.at[c, pl.ds(r0, RPT)])


@functools.lru_cache(maxsize=None)
def _sc_calls():
    mesh = plsc.VectorSubcoreMesh(core_axis_name="c", subcore_axis_name="s")
    cparams = pltpu.CompilerParams(use_tc_tiling_on_sc=False)
    ea_call = functools.partial(
        pl.kernel,
        out_type=jax.ShapeDtypeStruct((2, NPAD, DE), jnp.float32),
        mesh=mesh,
        compiler_params=cparams,
        scratch_types=[
            pltpu.VMEM((C, GRP), jnp.int32),
            [pltpu.VMEM((GRP, DE), jnp.float32)] * _NBUF,
            pltpu.VMEM_SHARED((NPAD, DE), jnp.float32),
            [pltpu.SemaphoreType.DMA] * _NBUF,
            [pltpu.SemaphoreType.DMA] * _NBUF,
        ],
    )(_sc_ea)
    scatter = functools.partial(
        pl.kernel,
        out_type=jax.ShapeDtypeStruct((2, NPAD, DH), jnp.bfloat16),
        mesh=mesh,
        compiler_params=cparams,
        scratch_types=[
            pltpu.VMEM((C, GRP), jnp.int32),
            pltpu.VMEM((C, GRP), jnp.int32),
            [pltpu.VMEM((GRP, DH), jnp.bfloat16)] * _NBUF,
            pltpu.VMEM_SHARED((NPAD, DH), jnp.bfloat16),
            [pltpu.SemaphoreType.DMA] * _NBUF,
            [pltpu.SemaphoreType.DMA] * _NBUF,
        ],
    )(_sc_scatter)
    return ea_call, scatter


def _agg(p_ref):
    return (p_ref[0].astype(jnp.float32) + p_ref[1].astype(jnp.float32))


def _tc1_body(x_ref, w1_ref, b1_ref, a1_ref, wn_ref, h0_ref, hn_ref):
    h = _dot(x_ref[...], w1_ref[...]) + b1_ref[...]
    h = _prelu(h, a1_ref[0, 0])
    h0_ref[...] = h
    hn_ref[...] = _dot(h, wn_ref[...]).astype(jnp.bfloat16)


def _tc2_body(h_ref, p_ref, e_ref, wr_ref, we_ref, b_ref, a_ref, wn_ref,
              h1_ref, hn_ref):
    eagg = e_ref[0] + e_ref[1]
    v = (_dot(h_ref[...], wr_ref[...]) + _agg(p_ref)
         + _dot(eagg, we_ref[...]) + b_ref[...])
    h1 = _prelu(v, a_ref[0, 0])
    h1_ref[...] = h1
    hn_ref[...] = _dot(h1, wn_ref[...]).astype(jnp.bfloat16)


def _tc3_body(h_ref, p_ref, e_ref, wr_ref, we_ref, b_ref, a_ref, bat_ref,
              w2_ref, b2_ref, out_ref, acc_ref, cnt_ref):
    i = pl.program_id(0)
    eagg = e_ref[0] + e_ref[1]
    v = (_dot(h_ref[...], wr_ref[...]) + _agg(p_ref)
         + _dot(eagg, we_ref[...]) + b_ref[...])
    h2 = _prelu(v, a_ref[0, 0])

    bb = bat_ref[0, 0, :]
    gi = lax.broadcasted_iota(jnp.int32, (G, BLK), 0)
    oh = (bb[None, :] == gi).astype(jnp.float32)

    @pl.when(i == 0)
    def _():
        acc_ref[...] = jnp.zeros((G, DH), jnp.float32)
        cnt_ref[...] = jnp.zeros((G, DH), jnp.float32)

    acc_ref[...] += _dot(oh, h2)
    cnt_ref[...] += jnp.broadcast_to(jnp.sum(oh, axis=1, keepdims=True), (G, DH))

    @pl.when(i == NB - 1)
    def _():
        sums = acc_ref[...]
        counts = cnt_ref[...][:, :1]
        hg = sums / jnp.maximum(counts, 1.0)
        nrm = jnp.sqrt(jnp.sum(hg * hg, axis=1, keepdims=True))
        hg = hg / jnp.maximum(nrm, 1e-12)
        out_ref[...] = _dot(hg, w2_ref[...]) + b2_ref[...]


def _full(shape):
    return pl.BlockSpec(shape, lambda i: tuple(0 for _ in shape))


def _rows(width):
    return pl.BlockSpec((BLK, width), lambda i: (i, 0))


_PSPEC = pl.BlockSpec((2, BLK, DH), lambda i: (0, i, 0))
_ESPEC = pl.BlockSpec((2, BLK, DE), lambda i: (0, i, 0))


def _tc1(x, w1, b1, a1, wn):
    return pl.pallas_call(
        _tc1_body,
        grid=(NB,),
        in_specs=[_rows(DF), _full((DF, DH)), _full((1, DH)), _full((1, 1)),
                  _full((DH, DH))],
        out_specs=[_rows(DH), _rows(DH)],
        out_shape=[jax.ShapeDtypeStruct((N, DH), jnp.float32),
                   jax.ShapeDtypeStruct((N, DH), jnp.bfloat16)],
    )(x, w1, b1, a1, wn)


def _tc2(h, p, e, wr, we, b, a, wn):
    return pl.pallas_call(
        _tc2_body,
        grid=(NB,),
        in_specs=[_rows(DH), _PSPEC, _ESPEC, _full((DH, DH)),
                  _full((DE, DH)), _full((1, DH)), _full((1, 1)),
                  _full((DH, DH))],
        out_specs=[_rows(DH), _rows(DH)],
        out_shape=[jax.ShapeDtypeStruct((N, DH), jnp.float32),
                   jax.ShapeDtypeStruct((N, DH), jnp.bfloat16)],
    )(h, p, e, wr, we, b, a, wn)


def _tc3(h, p, e, wr, we, b, a, bat, w2, b2):
    return pl.pallas_call(
        _tc3_body,
        grid=(NB,),
        in_specs=[_rows(DH), _PSPEC, _ESPEC, _full((DH, DH)),
                  _full((DE, DH)), _full((1, DH)), _full((1, 1)),
                  pl.BlockSpec((1, 1, BLK), lambda i: (i, 0, 0)),
                  _full((DH, DO)), _full((1, DO))],
        out_specs=_full((G, DO)),
        out_shape=jax.ShapeDtypeStruct((G, DO), jnp.float32),
        scratch_shapes=[pltpu.VMEM((G, DH), jnp.float32),
                        pltpu.VMEM((G, DH), jnp.float32)],
    )(h, p, e, wr, we, b, a, bat, w2, b2)


def kernel(x, edge_index, edge_attr, batch, fc1_W, fc1_b, a_fc1,
           gc1_Wr, gc1_Wn, gc1_We, gc1_b, a_gc1,
           gc2_Wr, gc2_Wn, gc2_We, gc2_b, a_gc2,
           fc2_W, fc2_b):
    src = edge_index[0]
    dst = edge_index[1]
    pad = EPAD - E
    src_b = jnp.concatenate(
        [src, jnp.zeros((pad,), jnp.int32)]).reshape(NTILES, C, GRP)
    dst_b = jnp.concatenate(
        [dst, jnp.full((pad,), NPAD - 1, jnp.int32)]).reshape(NTILES, C, GRP)
    ea_p = jnp.concatenate([edge_attr, jnp.zeros((pad, DE), jnp.float32)])
    zb = jnp.zeros((RPT, DH), jnp.bfloat16)
    z16 = jnp.zeros((RPT, DE), jnp.float32)
    bat3 = batch.reshape(NB, 1, BLK)

    b1 = fc1_b.reshape(1, DH)
    bg1 = gc1_b.reshape(1, DH)
    bg2 = gc2_b.reshape(1, DH)
    b2 = fc2_b.reshape(1, DO)
    a1 = a_fc1.reshape(1, 1)
    ag1 = a_gc1.reshape(1, 1)
    ag2 = a_gc2.reshape(1, 1)

    ea_call, scatter_call = _sc_calls()
    e16 = ea_call(ea_p, dst_b, z16)
    h0, hn1 = _tc1(x, fc1_W, b1, a1, gc1_Wn)
    p1 = scatter_call(hn1, src_b, dst_b, zb)
    h1, hn2 = _tc2(h0, p1, e16, gc1_Wr, gc1_We, bg1, ag1, gc2_Wn)
    p2 = scatter_call(hn2, src_b, dst_b, zb)
    out = _tc3(h1, p2, e16, gc2_Wr, gc2_We, bg2, ag2, bat3, fc2_W, b2)
    return out


# speedup vs baseline: 1.5226x; 1.2473x over previous
import functools

import jax
import jax.numpy as jnp
from jax import lax
from jax.experimental import pallas as pl
from jax.experimental.pallas import tpu as pltpu
from jax.experimental.pallas import tpu_sc as plsc

N = 10000
E = 320000
DF = 128
DH = 128
DO = 64
DE = 16
G = 64

HW = DH // 2
NTILES = 32
GRP = 128
C = 79
EPAD = NTILES * C * GRP
NPAD = 10240
RPT = NPAD // 16
BLK = 1000
NB = N // BLK

_HIGH = jax.lax.Precision.HIGHEST


def _dot(a, b):
    return jax.lax.dot(a, b, precision=_HIGH, preferred_element_type=jnp.float32)


def _prelu(v, a):
    return jnp.where(v >= 0, v, a * v)


_NBUF = 4


def _sc_scatter(hn, srcb, dstb, z, outp, src_v, dst_v, rows, acc, gs, ss):
    c = lax.axis_index("c")
    s = lax.axis_index("s")
    b = c * 16 + s
    pltpu.sync_copy(srcb.at[b], src_v)
    pltpu.sync_copy(dstb.at[b], dst_v)
    r0 = s * RPT
    pltpu.sync_copy(z, acc.at[pl.ds(r0, RPT)])

    def start_rows(g, p):
        pltpu.async_copy(hn.at[src_v.at[g]], rows[p], gs[p])

    def wait_rows(p):
        pltpu.make_async_copy(hn.at[src_v.at[0]], rows[p], gs[p]).wait()

    def start_scat(g, p):
        pltpu.async_copy(rows[p], acc.at[dst_v.at[g]], ss[p], add=True)

    def wait_scat(p):
        pltpu.make_async_copy(rows[p], acc.at[dst_v.at[0]], ss[p]).wait()

    start_rows(0, 0)
    start_rows(1, 1)
    plsc.subcore_barrier()

    def phase(g, p):
        wait_rows(p)

        @pl.when(g >= 2)
        def _():
            wait_scat((p - 2) % _NBUF)

        start_scat(g, p)

        @pl.when(g + 2 < C)
        def _():
            start_rows(g + 2, (p + 2) % _NBUF)

    def body(i, carry):
        j = i * _NBUF
        phase(j, 0)

        @pl.when(j + 1 < C)
        def _():
            phase(j + 1, 1)

        @pl.when(j + 2 < C)
        def _():
            phase(j + 2, 2)

        @pl.when(j + 3 < C)
        def _():
            phase(j + 3, 3)

        return carry

    lax.fori_loop(0, (C + _NBUF - 1) // _NBUF, body, 0)
    wait_scat((C - 1) % _NBUF)
    wait_scat((C - 2) % _NBUF)
    plsc.subcore_barrier()
    pltpu.sync_copy(acc.at[pl.ds(r0, RPT)], outp.at[c, pl.ds(r0, RPT)])


def _sc_ea(ea, dstb, z16, out16, dst_v, eab, acc16, egs, ess):
    c = lax.axis_index("c")
    s = lax.axis_index("s")
    b = c * 16 + s
    pltpu.sync_copy(dstb.at[b], dst_v)
    r0 = s * RPT
    pltpu.sync_copy(z16, acc16.at[pl.ds(r0, RPT)])
    ebase = b * (C * GRP)

    def start_ea(g, p):
        pltpu.async_copy(ea.at[pl.ds(ebase + g * GRP, GRP)], eab[p], egs[p])

    def wait_ea(p):
        pltpu.make_async_copy(ea.at[pl.ds(0, GRP)], eab[p], egs[p]).wait()

    def start_escat(g, p):
        pltpu.async_copy(eab[p], acc16.at[dst_v.at[g]], ess[p], add=True)

    def wait_escat(p):
        pltpu.make_async_copy(eab[p], acc16.at[dst_v.at[0]], ess[p]).wait()

    start_ea(0, 0)
    start_ea(1, 1)
    plsc.subcore_barrier()

    def phase(g, p):
        wait_ea(p)

        @pl.when(g >= 2)
        def _():
            wait_escat((p - 2) % _NBUF)

        start_escat(g, p)

        @pl.when(g + 2 < C)
        def _():
            start_ea(g + 2, (p + 2) % _NBUF)

    def body(i, carry):
        j = i * _NBUF
        phase(j, 0)

        @pl.when(j + 1 < C)
        def _():
            phase(j + 1, 1)

        @pl.when(j + 2 < C)
        def _():
            phase(j + 2, 2)

        @pl.when(j + 3 < C)
        def _():
            phase(j + 3, 3)

        return carry

    lax.fori_loop(0, (C + _NBUF - 1) // _NBUF, body, 0)
    wait_escat((C - 1) % _NBUF)
    wait_escat((C - 2) % _NBUF)
    plsc.subcore_barrier()
    pltpu.sync_copy(acc16.at[pl.ds(r0, RPT)], out16.at[c, pl.ds(r0, RPT)])


@functools.lru_cache(maxsize=None)
def _sc_calls():
    mesh = plsc.VectorSubcoreMesh(core_axis_name="c", subcore_axis_name="s")
    cparams = pltpu.CompilerParams(use_tc_tiling_on_sc=False)
    ea_call = functools.partial(
        pl.kernel,
        out_type=jax.ShapeDtypeStruct((2, NPAD, DE), jnp.float32),
        mesh=mesh,
        compiler_params=cparams,
        scratch_types=[
            pltpu.VMEM((C, GRP), jnp.int32),
            [pltpu.VMEM((GRP, DE), jnp.float32)] * _NBUF,
            pltpu.VMEM_SHARED((NPAD, DE), jnp.float32),
            [pltpu.SemaphoreType.DMA] * _NBUF,
            [pltpu.SemaphoreType.DMA] * _NBUF,
        ],
    )(_sc_ea)
    scatter = functools.partial(
        pl.kernel,
        out_type=jax.ShapeDtypeStruct((2, NPAD, DH), jnp.bfloat16),
        mesh=mesh,
        compiler_params=cparams,
        scratch_types=[
            pltpu.VMEM((C, GRP), jnp.int32),
            pltpu.VMEM((C, GRP), jnp.int32),
            [pltpu.VMEM((GRP, DH), jnp.bfloat16)] * _NBUF,
            pltpu.VMEM_SHARED((NPAD, DH), jnp.bfloat16),
            [pltpu.SemaphoreType.DMA] * _NBUF,
            [pltpu.SemaphoreType.DMA] * _NBUF,
        ],
    )(_sc_scatter)
    return ea_call, scatter


def _agg(p_ref):
    return (p_ref[0].astype(jnp.float32) + p_ref[1].astype(jnp.float32))


def _tc1_body(x_ref, w1_ref, b1_ref, a1_ref, wn_ref, h0_ref, hn_ref):
    h = _dot(x_ref[...], w1_ref[...]) + b1_ref[...]
    h = _prelu(h, a1_ref[0, 0])
    h0_ref[...] = h
    hn_ref[...] = _dot(h, wn_ref[...]).astype(jnp.bfloat16)


def _tc2_body(h_ref, p_ref, e_ref, wr_ref, we_ref, b_ref, a_ref, wn_ref,
              h1_ref, hn_ref):
    eagg = e_ref[0] + e_ref[1]
    v = (_dot(h_ref[...], wr_ref[...]) + _agg(p_ref)
         + _dot(eagg, we_ref[...]) + b_ref[...])
    h1 = _prelu(v, a_ref[0, 0])
    h1_ref[...] = h1
    hn_ref[...] = _dot(h1, wn_ref[...]).astype(jnp.bfloat16)


def _tc3_body(h_ref, p_ref, e_ref, wr_ref, we_ref, b_ref, a_ref, bat_ref,
              w2_ref, b2_ref, out_ref, acc_ref, cnt_ref):
    i = pl.program_id(0)
    eagg = e_ref[0] + e_ref[1]
    v = (_dot(h_ref[...], wr_ref[...]) + _agg(p_ref)
         + _dot(eagg, we_ref[...]) + b_ref[...])
    h2 = _prelu(v, a_ref[0, 0])

    bb = bat_ref[0, 0, :]
    gi = lax.broadcasted_iota(jnp.int32, (G, BLK), 0)
    oh = (bb[None, :] == gi).astype(jnp.float32)

    @pl.when(i == 0)
    def _():
        acc_ref[...] = jnp.zeros((G, DH), jnp.float32)
        cnt_ref[...] = jnp.zeros((G, DH), jnp.float32)

    acc_ref[...] += _dot(oh, h2)
    cnt_ref[...] += jnp.broadcast_to(jnp.sum(oh, axis=1, keepdims=True), (G, DH))

    @pl.when(i == NB - 1)
    def _():
        sums = acc_ref[...]
        counts = cnt_ref[...][:, :1]
        hg = sums / jnp.maximum(counts, 1.0)
        nrm = jnp.sqrt(jnp.sum(hg * hg, axis=1, keepdims=True))
        hg = hg / jnp.maximum(nrm, 1e-12)
        out_ref[...] = _dot(hg, w2_ref[...]) + b2_ref[...]


def _full(shape):
    return pl.BlockSpec(shape, lambda i: tuple(0 for _ in shape))


def _rows(width):
    return pl.BlockSpec((BLK, width), lambda i: (i, 0))


_PSPEC = pl.BlockSpec((2, BLK, DH), lambda i: (0, i, 0))
_ESPEC = pl.BlockSpec((2, BLK, DE), lambda i: (0, i, 0))


def _tc1(x, w1, b1, a1, wn):
    return pl.pallas_call(
        _tc1_body,
        grid=(NB,),
        in_specs=[_rows(DF), _full((DF, DH)), _full((1, DH)), _full((1, 1)),
                  _full((DH, DH))],
        out_specs=[_rows(DH), _rows(DH)],
        out_shape=[jax.ShapeDtypeStruct((N, DH), jnp.float32),
                   jax.ShapeDtypeStruct((N, DH), jnp.bfloat16)],
    )(x, w1, b1, a1, wn)


def _tc2(h, p, e, wr, we, b, a, wn):
    return pl.pallas_call(
        _tc2_body,
        grid=(NB,),
        in_specs=[_rows(DH), _PSPEC, _ESPEC, _full((DH, DH)),
                  _full((DE, DH)), _full((1, DH)), _full((1, 1)),
                  _full((DH, DH))],
        out_specs=[_rows(DH), _rows(DH)],
        out_shape=[jax.ShapeDtypeStruct((N, DH), jnp.float32),
                   jax.ShapeDtypeStruct((N, DH), jnp.bfloat16)],
    )(h, p, e, wr, we, b, a, wn)


def _tc3(h, p, e, wr, we, b, a, bat, w2, b2):
    return pl.pallas_call(
        _tc3_body,
        grid=(NB,),
        in_specs=[_rows(DH), _PSPEC, _ESPEC, _full((DH, DH)),
                  _full((DE, DH)), _full((1, DH)), _full((1, 1)),
                  pl.BlockSpec((1, 1, BLK), lambda i: (i, 0, 0)),
                  _full((DH, DO)), _full((1, DO))],
        out_specs=_full((G, DO)),
        out_shape=jax.ShapeDtypeStruct((G, DO), jnp.float32),
        scratch_shapes=[pltpu.VMEM((G, DH), jnp.float32),
                        pltpu.VMEM((G, DH), jnp.float32)],
    )(h, p, e, wr, we, b, a, bat, w2, b2)


def kernel(x, edge_index, edge_attr, batch, fc1_W, fc1_b, a_fc1,
           gc1_Wr, gc1_Wn, gc1_We, gc1_b, a_gc1,
           gc2_Wr, gc2_Wn, gc2_We, gc2_b, a_gc2,
           fc2_W, fc2_b):
    src = edge_index[0]
    dst = edge_index[1]
    pad = EPAD - E
    cyc = jnp.arange(pad, dtype=jnp.int32)
    src_b = jnp.concatenate([src, cyc % N]).reshape(NTILES, C, GRP)
    dst_b = jnp.concatenate(
        [dst, N + (cyc % (NPAD - N))]).reshape(NTILES, C, GRP)
    ea_p = jnp.concatenate([edge_attr, jnp.zeros((pad, DE), jnp.float32)])
    zb = jnp.zeros((RPT, DH), jnp.bfloat16)
    z16 = jnp.zeros((RPT, DE), jnp.float32)
    bat3 = batch.reshape(NB, 1, BLK)

    b1 = fc1_b.reshape(1, DH)
    bg1 = gc1_b.reshape(1, DH)
    bg2 = gc2_b.reshape(1, DH)
    b2 = fc2_b.reshape(1, DO)
    a1 = a_fc1.reshape(1, 1)
    ag1 = a_gc1.reshape(1, 1)
    ag2 = a_gc2.reshape(1, 1)

    ea_call, scatter_call = _sc_calls()
    e16 = ea_call(ea_p, dst_b, z16)
    h0, hn1 = _tc1(x, fc1_W, b1, a1, gc1_Wn)
    p1 = scatter_call(hn1, src_b, dst_b, zb)
    h1, hn2 = _tc2(h0, p1, e16, gc1_Wr, gc1_We, bg1, ag1, gc2_Wn)
    p2 = scatter_call(hn2, src_b, dst_b, zb)
    out = _tc3(h1, p2, e16, gc2_Wr, gc2_We, bg2, ag2, bat3, fc2_W, b2)
    return out


# speedup vs baseline: 1.5878x; 1.0428x over previous
import functools

import jax
import jax.numpy as jnp
from jax import lax
from jax.experimental import pallas as pl
from jax.experimental.pallas import tpu as pltpu
from jax.experimental.pallas import tpu_sc as plsc

N = 10000
E = 320000
DF = 128
DH = 128
DO = 64
DE = 16
G = 64

HW = DH // 2
NTILES = 32
GRP = 128
C = 79
EPAD = NTILES * C * GRP
NPAD = 10240
RPT = NPAD // 16
BLK = 1000
NB = N // BLK

_HIGH = jax.lax.Precision.HIGHEST


def _dot(a, b):
    return jax.lax.dot(a, b, precision=_HIGH, preferred_element_type=jnp.float32)


def _prelu(v, a):
    return jnp.where(v >= 0, v, a * v)


_NBUF = 6


def _sc_scatter(hn, srcb, dstb, z, outp, src_v, dst_v, rows, acc, gs, ss):
    c = lax.axis_index("c")
    s = lax.axis_index("s")
    b = c * 16 + s
    pltpu.sync_copy(srcb.at[b], src_v)
    pltpu.sync_copy(dstb.at[b], dst_v)
    r0 = s * RPT
    pltpu.sync_copy(z, acc.at[pl.ds(r0, RPT)])

    def start_rows(g, p):
        pltpu.async_copy(hn.at[src_v.at[g]], rows[p], gs[p])

    def wait_rows(p):
        pltpu.make_async_copy(hn.at[src_v.at[0]], rows[p], gs[p]).wait()

    def start_scat(g, p):
        pltpu.async_copy(rows[p], acc.at[dst_v.at[g]], ss[p], add=True)

    def wait_scat(p):
        pltpu.make_async_copy(rows[p], acc.at[dst_v.at[0]], ss[p]).wait()

    depth = _NBUF - 2
    for k in range(depth):
        start_rows(k, k)
    plsc.subcore_barrier()

    def phase(g, p):
        wait_rows(p)

        @pl.when(g >= 2)
        def _():
            wait_scat((p - 2) % _NBUF)

        start_scat(g, p)

        @pl.when(g + depth < C)
        def _():
            start_rows(g + depth, (p + depth) % _NBUF)

    def body(i, carry):
        j = i * _NBUF
        phase(j, 0)
        for p in range(1, _NBUF):
            @pl.when(j + p < C)
            def _(p=p):
                phase(j + p, p)
        return carry

    lax.fori_loop(0, (C + _NBUF - 1) // _NBUF, body, 0)
    wait_scat((C - 1) % _NBUF)
    wait_scat((C - 2) % _NBUF)
    plsc.subcore_barrier()
    pltpu.sync_copy(acc.at[pl.ds(r0, RPT)], outp.at[c, pl.ds(r0, RPT)])


def _sc_ea(ea, dstb, z16, out16, dst_v, eab, acc16, egs, ess):
    c = lax.axis_index("c")
    s = lax.axis_index("s")
    b = c * 16 + s
    pltpu.sync_copy(dstb.at[b], dst_v)
    r0 = s * RPT
    pltpu.sync_copy(z16, acc16.at[pl.ds(r0, RPT)])
    ebase = b * (C * GRP)

    def start_ea(g, p):
        pltpu.async_copy(ea.at[pl.ds(ebase + g * GRP, GRP)], eab[p], egs[p])

    def wait_ea(p):
        pltpu.make_async_copy(ea.at[pl.ds(0, GRP)], eab[p], egs[p]).wait()

    def start_escat(g, p):
        pltpu.async_copy(eab[p], acc16.at[dst_v.at[g]], ess[p], add=True)

    def wait_escat(p):
        pltpu.make_async_copy(eab[p], acc16.at[dst_v.at[0]], ess[p]).wait()

    depth = _NBUF - 2
    for k in range(depth):
        start_ea(k, k)
    plsc.subcore_barrier()

    def phase(g, p):
        wait_ea(p)

        @pl.when(g >= 2)
        def _():
            wait_escat((p - 2) % _NBUF)

        start_escat(g, p)

        @pl.when(g + depth < C)
        def _():
            start_ea(g + depth, (p + depth) % _NBUF)

    def body(i, carry):
        j = i * _NBUF
        phase(j, 0)
        for p in range(1, _NBUF):
            @pl.when(j + p < C)
            def _(p=p):
                phase(j + p, p)
        return carry

    lax.fori_loop(0, (C + _NBUF - 1) // _NBUF, body, 0)
    wait_escat((C - 1) % _NBUF)
    wait_escat((C - 2) % _NBUF)
    plsc.subcore_barrier()
    pltpu.sync_copy(acc16.at[pl.ds(r0, RPT)], out16.at[c, pl.ds(r0, RPT)])


@functools.lru_cache(maxsize=None)
def _sc_calls():
    mesh = plsc.VectorSubcoreMesh(core_axis_name="c", subcore_axis_name="s")
    cparams = pltpu.CompilerParams(use_tc_tiling_on_sc=False)
    ea_call = functools.partial(
        pl.kernel,
        out_type=jax.ShapeDtypeStruct((2, NPAD, DE), jnp.float32),
        mesh=mesh,
        compiler_params=cparams,
        scratch_types=[
            pltpu.VMEM((C, GRP), jnp.int32),
            [pltpu.VMEM((GRP, DE), jnp.float32)] * _NBUF,
            pltpu.VMEM_SHARED((NPAD, DE), jnp.float32),
            [pltpu.SemaphoreType.DMA] * _NBUF,
            [pltpu.SemaphoreType.DMA] * _NBUF,
        ],
    )(_sc_ea)
    scatter = functools.partial(
        pl.kernel,
        out_type=jax.ShapeDtypeStruct((2, NPAD, DH), jnp.bfloat16),
        mesh=mesh,
        compiler_params=cparams,
        scratch_types=[
            pltpu.VMEM((C, GRP), jnp.int32),
            pltpu.VMEM((C, GRP), jnp.int32),
            [pltpu.VMEM((GRP, DH), jnp.bfloat16)] * _NBUF,
            pltpu.VMEM_SHARED((NPAD, DH), jnp.bfloat16),
            [pltpu.SemaphoreType.DMA] * _NBUF,
            [pltpu.SemaphoreType.DMA] * _NBUF,
        ],
    )(_sc_scatter)
    return ea_call, scatter


def _agg(p_ref):
    return (p_ref[0].astype(jnp.float32) + p_ref[1].astype(jnp.float32))


def _tc1_body(x_ref, w1_ref, b1_ref, a1_ref, wn_ref, h0_ref, hn_ref):
    h = _dot(x_ref[...], w1_ref[...]) + b1_ref[...]
    h = _prelu(h, a1_ref[0, 0])
    h0_ref[...] = h
    hn_ref[...] = _dot(h, wn_ref[...]).astype(jnp.bfloat16)


def _tc2_body(h_ref, p_ref, e_ref, wr_ref, we_ref, b_ref, a_ref, wn_ref,
              h1_ref, hn_ref):
    eagg = e_ref[0] + e_ref[1]
    v = (_dot(h_ref[...], wr_ref[...]) + _agg(p_ref)
         + _dot(eagg, we_ref[...]) + b_ref[...])
    h1 = _prelu(v, a_ref[0, 0])
    h1_ref[...] = h1
    hn_ref[...] = _dot(h1, wn_ref[...]).astype(jnp.bfloat16)


def _tc3_body(h_ref, p_ref, e_ref, wr_ref, we_ref, b_ref, a_ref, bat_ref,
              w2_ref, b2_ref, out_ref, acc_ref, cnt_ref):
    i = pl.program_id(0)
    eagg = e_ref[0] + e_ref[1]
    v = (_dot(h_ref[...], wr_ref[...]) + _agg(p_ref)
         + _dot(eagg, we_ref[...]) + b_ref[...])
    h2 = _prelu(v, a_ref[0, 0])

    bb = bat_ref[0, 0, :]
    gi = lax.broadcasted_iota(jnp.int32, (G, BLK), 0)
    oh = (bb[None, :] == gi).astype(jnp.float32)

    @pl.when(i == 0)
    def _():
        acc_ref[...] = jnp.zeros((G, DH), jnp.float32)
        cnt_ref[...] = jnp.zeros((G, DH), jnp.float32)

    acc_ref[...] += _dot(oh, h2)
    cnt_ref[...] += jnp.broadcast_to(jnp.sum(oh, axis=1, keepdims=True), (G, DH))

    @pl.when(i == NB - 1)
    def _():
        sums = acc_ref[...]
        counts = cnt_ref[...][:, :1]
        hg = sums / jnp.maximum(counts, 1.0)
        nrm = jnp.sqrt(jnp.sum(hg * hg, axis=1, keepdims=True))
        hg = hg / jnp.maximum(nrm, 1e-12)
        out_ref[...] = _dot(hg, w2_ref[...]) + b2_ref[...]


def _full(shape):
    return pl.BlockSpec(shape, lambda i: tuple(0 for _ in shape))


def _rows(width):
    return pl.BlockSpec((BLK, width), lambda i: (i, 0))


_PSPEC = pl.BlockSpec((2, BLK, DH), lambda i: (0, i, 0))
_ESPEC = pl.BlockSpec((2, BLK, DE), lambda i: (0, i, 0))


def _tc1(x, w1, b1, a1, wn):
    return pl.pallas_call(
        _tc1_body,
        grid=(NB,),
        in_specs=[_rows(DF), _full((DF, DH)), _full((1, DH)), _full((1, 1)),
                  _full((DH, DH))],
        out_specs=[_rows(DH), _rows(DH)],
        out_shape=[jax.ShapeDtypeStruct((N, DH), jnp.float32),
                   jax.ShapeDtypeStruct((N, DH), jnp.bfloat16)],
    )(x, w1, b1, a1, wn)


def _tc2(h, p, e, wr, we, b, a, wn):
    return pl.pallas_call(
        _tc2_body,
        grid=(NB,),
        in_specs=[_rows(DH), _PSPEC, _ESPEC, _full((DH, DH)),
                  _full((DE, DH)), _full((1, DH)), _full((1, 1)),
                  _full((DH, DH))],
        out_specs=[_rows(DH), _rows(DH)],
        out_shape=[jax.ShapeDtypeStruct((N, DH), jnp.float32),
                   jax.ShapeDtypeStruct((N, DH), jnp.bfloat16)],
    )(h, p, e, wr, we, b, a, wn)


def _tc3(h, p, e, wr, we, b, a, bat, w2, b2):
    return pl.pallas_call(
        _tc3_body,
        grid=(NB,),
        in_specs=[_rows(DH), _PSPEC, _ESPEC, _full((DH, DH)),
                  _full((DE, DH)), _full((1, DH)), _full((1, 1)),
                  pl.BlockSpec((1, 1, BLK), lambda i: (i, 0, 0)),
                  _full((DH, DO)), _full((1, DO))],
        out_specs=_full((G, DO)),
        out_shape=jax.ShapeDtypeStruct((G, DO), jnp.float32),
        scratch_shapes=[pltpu.VMEM((G, DH), jnp.float32),
                        pltpu.VMEM((G, DH), jnp.float32)],
    )(h, p, e, wr, we, b, a, bat, w2, b2)


def kernel(x, edge_index, edge_attr, batch, fc1_W, fc1_b, a_fc1,
           gc1_Wr, gc1_Wn, gc1_We, gc1_b, a_gc1,
           gc2_Wr, gc2_Wn, gc2_We, gc2_b, a_gc2,
           fc2_W, fc2_b):
    src = edge_index[0]
    dst = edge_index[1]
    pad = EPAD - E
    cyc = jnp.arange(pad, dtype=jnp.int32)
    src_b = jnp.concatenate([src, cyc % N]).reshape(NTILES, C, GRP)
    dst_b = jnp.concatenate(
        [dst, N + (cyc % (NPAD - N))]).reshape(NTILES, C, GRP)
    ea_p = jnp.concatenate([edge_attr, jnp.zeros((pad, DE), jnp.float32)])
    zb = jnp.zeros((RPT, DH), jnp.bfloat16)
    z16 = jnp.zeros((RPT, DE), jnp.float32)
    bat3 = batch.reshape(NB, 1, BLK)

    b1 = fc1_b.reshape(1, DH)
    bg1 = gc1_b.reshape(1, DH)
    bg2 = gc2_b.reshape(1, DH)
    b2 = fc2_b.reshape(1, DO)
    a1 = a_fc1.reshape(1, 1)
    ag1 = a_gc1.reshape(1, 1)
    ag2 = a_gc2.reshape(1, 1)

    ea_call, scatter_call = _sc_calls()
    e16 = ea_call(ea_p, dst_b, z16)
    h0, hn1 = _tc1(x, fc1_W, b1, a1, gc1_Wn)
    p1 = scatter_call(hn1, src_b, dst_b, zb)
    h1, hn2 = _tc2(h0, p1, e16, gc1_Wr, gc1_We, bg1, ag1, gc2_Wn)
    p2 = scatter_call(hn2, src_b, dst_b, zb)
    out = _tc3(h1, p2, e16, gc2_Wr, gc2_We, bg2, ag2, bat3, fc2_W, b2)
    return out
